# baseline (speedup 1.0000x reference)
import jax
import jax.numpy as jnp
from jax.experimental import pallas as pl
from jax.experimental.pallas import tpu as pltpu

_SQRT_HALF = 0.7071067811865476
_NB = 4


def _mlp_kernel(x_ref, w1_ref, b1_ref, w2_ref, b2_ref, o_ref):
    for i in range(_NB):
        x = x_ref[i]
        h = jnp.dot(w1_ref[...], x, preferred_element_type=jnp.float32)
        h = h + b1_ref[...]
        g = 0.5 * h * (1.0 + jax.lax.erf(h * _SQRT_HALF))
        y = jnp.dot(w2_ref[...], g, preferred_element_type=jnp.float32)
        o_ref[i] = (y + b2_ref[...]).astype(jnp.bfloat16)


def kernel(x, w1, b1, w2, b2):
    B, Cin, H, W = x.shape
    hidden = w1.shape[0]
    Cout = w2.shape[0]
    HW = H * W

    x3 = x.reshape(B, Cin, HW)

    full2d = lambda shape: pl.BlockSpec(shape, lambda b: (0, 0))
    flops = 2 * B * HW * (Cin * hidden + hidden * Cout)
    cost = pl.CostEstimate(flops=flops,
                           transcendentals=B * HW * hidden,
                           bytes_accessed=4 * B * HW * Cin + 2 * B * HW * Cout)

    out3 = pl.pallas_call(
        _mlp_kernel,
        out_shape=jax.ShapeDtypeStruct((B, Cout, HW), jnp.bfloat16),
        grid=(B // _NB,),
        in_specs=[
            pl.BlockSpec((_NB, Cin, HW), lambda b: (b, 0, 0)),
            full2d((hidden, Cin)),
            full2d((hidden, 1)),
            full2d((Cout, hidden)),
            full2d((Cout, 1)),
        ],
        out_specs=pl.BlockSpec((_NB, Cout, HW), lambda b: (b, 0, 0)),
        compiler_params=pltpu.CompilerParams(
            dimension_semantics=("parallel",),
        ),
        cost_estimate=cost,
    )(x3, w1, b1, w2, b2)

    return out3.astype(jnp.float32).reshape(B, Cout, H, W)


# speedup vs baseline: 1.0128x; 1.0128x over previous
import jax
import jax.numpy as jnp
from jax.experimental import pallas as pl
from jax.experimental.pallas import tpu as pltpu

_SQRT_HALF = 0.7071067811865476
_NB = 2


def _mlp_kernel(x_ref, w1_ref, b1_ref, w2_ref, b2_ref, o_ref):
    for i in range(_NB):
        x = x_ref[i]
        h = jnp.dot(w1_ref[...], x, preferred_element_type=jnp.float32)
        h = h + b1_ref[...]
        g = 0.5 * h * (1.0 + jax.lax.erf(h * _SQRT_HALF))
        y = jnp.dot(w2_ref[...], g, preferred_element_type=jnp.float32)
        o_ref[i] = (y + b2_ref[...]).astype(jnp.bfloat16)


def kernel(x, w1, b1, w2, b2):
    B, Cin, H, W = x.shape
    hidden = w1.shape[0]
    Cout = w2.shape[0]
    HW = H * W

    x3 = x.reshape(B, Cin, HW)

    full2d = lambda shape: pl.BlockSpec(shape, lambda b: (0, 0))
    flops = 2 * B * HW * (Cin * hidden + hidden * Cout)
    cost = pl.CostEstimate(flops=flops,
                           transcendentals=B * HW * hidden,
                           bytes_accessed=4 * B * HW * Cin + 2 * B * HW * Cout)

    out3 = pl.pallas_call(
        _mlp_kernel,
        out_shape=jax.ShapeDtypeStruct((B, Cout, HW), jnp.bfloat16),
        grid=(B // _NB,),
        in_specs=[
            pl.BlockSpec((_NB, Cin, HW), lambda b: (b, 0, 0)),
            full2d((hidden, Cin)),
            full2d((hidden, 1)),
            full2d((Cout, hidden)),
            full2d((Cout, 1)),
        ],
        out_specs=pl.BlockSpec((_NB, Cout, HW), lambda b: (b, 0, 0)),
        compiler_params=pltpu.CompilerParams(
            dimension_semantics=("parallel",),
        ),
        cost_estimate=cost,
    )(x3, w1, b1, w2, b2)

    return out3.astype(jnp.float32).reshape(B, Cout, H, W)
